# baseline (speedup 1.0000x reference)
import functools

import jax
import jax.numpy as jnp
from jax import lax
from jax.experimental import pallas as pl
from jax.experimental.pallas import tpu as pltpu
from jax.experimental.pallas import tpu_sc as plsc

_N = 2048
_K = 16
_RB = 256
_NEG = -3.0e38

_NTILES = 32
_RPT = _N // _NTILES
_CHUNK = 8


def _normalize_body(x_ref, o_ref):
    x = x_ref[...]
    norms = jnp.sqrt(jnp.sum(x * x, axis=1, keepdims=True))
    o_ref[...] = x / jnp.maximum(norms, 1e-12)


def _simtopk_body(fb_ref, feats_ref, xb_ref, w1_ref, idx_ref, y1_ref,
                  sim_scr):
    i = pl.program_id(0)
    nb = pl.num_programs(0) - 1

    @pl.when(i < nb)
    def _compute():
        sim = lax.dot_general(fb_ref[...], feats_ref[...],
                              (((1,), (1,)), ((), ())),
                              preferred_element_type=jnp.float32)
        sim_scr[i % 2] = sim
        y1_ref[...] = lax.dot_general(xb_ref[...], w1_ref[...],
                                      (((1,), (1,)), ((), ())),
                                      preferred_element_type=jnp.float32)

    @pl.when(i > 0)
    def _extract():
        v = sim_scr[(i - 1) % 2]
        col = lax.broadcasted_iota(jnp.int32, v.shape, 1)
        outs = []
        for _ in range(_K):
            m = jnp.max(v, axis=1, keepdims=True)
            sel = jnp.min(jnp.where(v == m, col, _N), axis=1, keepdims=True)
            outs.append(sel)
            v = jnp.where(col == sel, _NEG, v)
        idx_ref[...] = jnp.concatenate(outs, axis=1)


def _dstate_body(state_ref, d_ref, thr_ref):
    st = state_ref[...]
    dega = jnp.sum((st >= 1).astype(jnp.float32), axis=1, keepdims=True)
    degp = jnp.sum((st == 2).astype(jnp.float32), axis=1, keepdims=True)
    s = jnp.sum(degp)
    use_pruned = s >= 2.0 * _N
    deg = jnp.where(use_pruned, degp, dega) + 1.0
    d_ref[...] = 1.0 / jnp.sqrt(jnp.maximum(deg, 1e-10))
    thr_ref[...] = jnp.where(use_pruned,
                             jnp.full((1, _K), 2, jnp.int32),
                             jnp.full((1, _K), 1, jnp.int32))


def _layer_body(ab_ref, y_ref, b_ref, w2_ref, o_ref):
    s = lax.dot_general(ab_ref[...], y_ref[...], (((1,), (0,)), ((), ())),
                        preferred_element_type=jnp.float32)
    h = jnp.maximum(s + b_ref[...], 0.0)
    o_ref[...] = lax.dot_general(h, w2_ref[...], (((1,), (1,)), ((), ())),
                                 preferred_element_type=jnp.float32)


def _out_body(ab_ref, y2_ref, b2_ref, o_ref):
    s = lax.dot_general(ab_ref[...], y2_ref[...], (((1,), (0,)), ((), ())),
                        preferred_element_type=jnp.float32)
    o_ref[...] = s + b2_ref[...]


_MESH = plsc.VectorSubcoreMesh(core_axis_name="c", subcore_axis_name="s")


def _sc_wid():
    return lax.axis_index("c") * 16 + lax.axis_index("s")


@functools.partial(
    pl.kernel,
    mesh=_MESH,
    compiler_params=pltpu.CompilerParams(needs_layout_passes=False),
    out_type=jax.ShapeDtypeStruct((_N * _K,), jnp.int32),
    scratch_types=[
        pltpu.VMEM((_N * _K,), jnp.int32),
        pltpu.VMEM((_N,), jnp.int32),
        pltpu.VMEM((_RPT * _K,), jnp.int32),
    ],
)
def _sc_edge_state(topk_hbm, state_hbm, topk_v, fl_v, st_v):
    wid = _sc_wid()
    row0 = wid * _RPT
    pltpu.sync_copy(topk_hbm, topk_v)
    zeros = jnp.zeros((_K,), jnp.int32)
    ones = jnp.full((_K,), 1, jnp.int32)

    def zflag_body(z, _):
        fl_v[pl.ds(z * _K, _K)] = zeros
        return 0

    lax.fori_loop(0, _N // _K, zflag_body, 0)

    def row_body(r, _):
        i = row0 + r
        ivec = jnp.full((_K,), 1, jnp.int32) * i
        a = topk_v[pl.ds(i * _K, _K)]
        valid = a != ivec
        plsc.store_scatter(fl_v, [a], ones, mask=valid)
        cnt = jnp.zeros((_K,), jnp.int32)
        for u in range(_K):
            g = plsc.load_gather(topk_v, [a * _K + u])
            bit = plsc.load_gather(fl_v, [g])
            cnt = cnt + jnp.where(g != a, bit, 0)
        plsc.store_scatter(fl_v, [a], zeros, mask=valid)
        state = jnp.where(valid,
                          1 + (cnt >= 2).astype(jnp.int32),
                          jnp.zeros((_K,), jnp.int32))
        st_v[pl.ds(r * _K, _K)] = state
        return 0

    lax.fori_loop(0, _RPT, row_body, 0)
    pltpu.sync_copy(st_v, state_hbm.at[pl.ds(row0 * _K, _RPT * _K)])


@functools.partial(
    pl.kernel,
    mesh=_MESH,
    compiler_params=pltpu.CompilerParams(needs_layout_passes=False),
    out_type=jax.ShapeDtypeStruct((_N * _N,), jnp.float32),
    scratch_types=[
        pltpu.VMEM((_RPT * _K,), jnp.int32),
        pltpu.VMEM((_RPT * _K,), jnp.int32),
        pltpu.VMEM((_N,), jnp.float32),
        pltpu.VMEM((_K,), jnp.int32),
        pltpu.VMEM((_CHUNK * _N,), jnp.float32),
    ],
)
def _sc_scatter_adj(topk_hbm, state_hbm, d_hbm, thr_hbm, a_hbm,
                    topk_v, st_v, d_v, thr_v, buf_v):
    wid = _sc_wid()
    row0 = wid * _RPT
    pltpu.sync_copy(topk_hbm.at[pl.ds(row0 * _K, _RPT * _K)], topk_v)
    pltpu.sync_copy(state_hbm.at[pl.ds(row0 * _K, _RPT * _K)], st_v)
    pltpu.sync_copy(d_hbm, d_v)
    pltpu.sync_copy(thr_hbm, thr_v)
    thr = thr_v[...]
    lane0 = lax.broadcasted_iota(jnp.int32, (_K,), 0) == 0
    zeros = jnp.zeros((_K,), jnp.float32)

    def zero_body(z, _):
        buf_v[pl.ds(z * _K, _K)] = zeros
        return 0

    lax.fori_loop(0, _CHUNK * _N // _K, zero_body, 0)

    def chunk_body(c, _):
        def row_pass(r, write):
            lr = c * _CHUNK + r
            i = row0 + lr
            ivec = jnp.full((_K,), 1, jnp.int32) * i
            a = topk_v[pl.ds(lr * _K, _K)]
            st = st_v[pl.ds(lr * _K, _K)]
            keep = jnp.logical_and(st >= thr, a != ivec)
            da = plsc.load_gather(d_v, [a])
            di = plsc.load_gather(d_v, [ivec])
            off = r * _N
            if write:
                plsc.store_scatter(buf_v, [a + off], di * da, mask=keep)
                plsc.store_scatter(buf_v, [ivec + off], di * di, mask=lane0)
            else:
                plsc.store_scatter(buf_v, [a + off], zeros, mask=keep)
                plsc.store_scatter(buf_v, [ivec + off], zeros, mask=lane0)
            return 0

        for r in range(_CHUNK):
            row_pass(r, True)
        pltpu.sync_copy(
            buf_v, a_hbm.at[pl.ds((row0 + c * _CHUNK) * _N, _CHUNK * _N)])
        for r in range(_CHUNK):
            row_pass(r, False)
        return 0

    lax.fori_loop(0, _RPT // _CHUNK, chunk_body, 0)


def kernel(x, W1, b1, W2, b2):
    n, din = x.shape
    hid = W1.shape[0]
    dout = W2.shape[0]
    nb = n // _RB

    feats = pl.pallas_call(
        _normalize_body,
        grid=(nb,),
        in_specs=[pl.BlockSpec((_RB, din), lambda i: (i, 0))],
        out_specs=pl.BlockSpec((_RB, din), lambda i: (i, 0)),
        out_shape=jax.ShapeDtypeStruct((n, din), jnp.float32),
    )(x)

    last = nb - 1
    topk, y1 = pl.pallas_call(
        _simtopk_body,
        grid=(nb + 1,),
        in_specs=[pl.BlockSpec((_RB, din), lambda i: (jnp.minimum(i, last), 0)),
                  pl.BlockSpec((n, din), lambda i: (0, 0)),
                  pl.BlockSpec((_RB, din), lambda i: (jnp.minimum(i, last), 0)),
                  pl.BlockSpec((hid, din), lambda i: (0, 0))],
        out_specs=[pl.BlockSpec((_RB, _K),
                                lambda i: (jnp.maximum(i - 1, 0), 0)),
                   pl.BlockSpec((_RB, hid),
                                lambda i: (jnp.minimum(i, last), 0))],
        out_shape=[jax.ShapeDtypeStruct((n, _K), jnp.int32),
                   jax.ShapeDtypeStruct((n, hid), jnp.float32)],
        scratch_shapes=[pltpu.VMEM((2, _RB, _N), jnp.float32)],
    )(feats, feats, x, W1)

    topk_flat = topk.reshape(n * _K)
    state_flat = _sc_edge_state(topk_flat)
    state = state_flat.reshape(n, _K)

    d, thr = pl.pallas_call(
        _dstate_body,
        in_specs=[pl.BlockSpec((_N, _K), lambda: (0, 0))],
        out_specs=[pl.BlockSpec((_N, 1), lambda: (0, 0)),
                   pl.BlockSpec((1, _K), lambda: (0, 0))],
        out_shape=[jax.ShapeDtypeStruct((n, 1), jnp.float32),
                   jax.ShapeDtypeStruct((1, _K), jnp.int32)],
    )(state)

    a_flat = _sc_scatter_adj(topk_flat, state_flat, d.reshape(n),
                             thr.reshape(_K))
    a_norm = a_flat.reshape(n, n)

    y2 = pl.pallas_call(
        _layer_body,
        grid=(nb,),
        in_specs=[pl.BlockSpec((_RB, _N), lambda i: (i, 0)),
                  pl.BlockSpec((n, hid), lambda i: (0, 0)),
                  pl.BlockSpec((1, hid), lambda i: (0, 0)),
                  pl.BlockSpec((dout, hid), lambda i: (0, 0))],
        out_specs=pl.BlockSpec((_RB, dout), lambda i: (i, 0)),
        out_shape=jax.ShapeDtypeStruct((n, dout), jnp.float32),
    )(a_norm, y1, b1.reshape(1, hid), W2)

    out = pl.pallas_call(
        _out_body,
        grid=(nb,),
        in_specs=[pl.BlockSpec((_RB, _N), lambda i: (i, 0)),
                  pl.BlockSpec((n, dout), lambda i: (0, 0)),
                  pl.BlockSpec((1, dout), lambda i: (0, 0))],
        out_specs=pl.BlockSpec((_RB, dout), lambda i: (i, 0)),
        out_shape=jax.ShapeDtypeStruct((n, dout), jnp.float32),
    )(a_norm, y2, b2.reshape(1, dout))

    return out


# speedup vs baseline: 1.0261x; 1.0261x over previous
import functools

import numpy as np

import jax
import jax.numpy as jnp
from jax import lax
from jax.experimental import pallas as pl
from jax.experimental.pallas import tpu as pltpu
from jax.experimental.pallas import tpu_sc as plsc

_N = 2048
_K = 16
_RB = 256
_NEG = -3.0e38

_NTILES = 32
_RPT = _N // _NTILES
_CHUNK = 8

_DTBL = (np.maximum(np.arange(32), 1).astype(np.float64) ** -0.5
         ).astype(np.float32)


def _normalize_body(x_ref, o_ref):
    x = x_ref[...]
    norms = jnp.sqrt(jnp.sum(x * x, axis=1, keepdims=True))
    o_ref[...] = x / jnp.maximum(norms, 1e-12)


def _simtopk_body(fb_ref, feats_ref, xb_ref, w1_ref, idx_ref, y1_ref,
                  sim_scr):
    i = pl.program_id(0)
    nb = pl.num_programs(0) - 1

    @pl.when(i < nb)
    def _compute():
        sim = lax.dot_general(fb_ref[...], feats_ref[...],
                              (((1,), (1,)), ((), ())),
                              preferred_element_type=jnp.float32)
        sim_scr[i % 2] = sim
        y1_ref[...] = lax.dot_general(xb_ref[...], w1_ref[...],
                                      (((1,), (1,)), ((), ())),
                                      preferred_element_type=jnp.float32)

    @pl.when(i > 0)
    def _extract():
        v = sim_scr[(i - 1) % 2]
        col = lax.broadcasted_iota(jnp.int32, v.shape, 1)
        outs = []
        for _ in range(_K):
            m = jnp.max(v, axis=1, keepdims=True)
            sel = jnp.min(jnp.where(v == m, col, _N), axis=1, keepdims=True)
            outs.append(sel)
            v = jnp.where(col == sel, _NEG, v)
        idx_ref[...] = jnp.concatenate(outs, axis=1)


def _layer_body(ab_ref, y_ref, b_ref, w2_ref, o_ref):
    s = lax.dot_general(ab_ref[...], y_ref[...], (((1,), (0,)), ((), ())),
                        preferred_element_type=jnp.float32)
    h = jnp.maximum(s + b_ref[...], 0.0)
    o_ref[...] = lax.dot_general(h, w2_ref[...], (((1,), (1,)), ((), ())),
                                 preferred_element_type=jnp.float32)


def _out_body(ab_ref, y2_ref, b2_ref, o_ref):
    s = lax.dot_general(ab_ref[...], y2_ref[...], (((1,), (0,)), ((), ())),
                        preferred_element_type=jnp.float32)
    o_ref[...] = s + b2_ref[...]


_MESH = plsc.VectorSubcoreMesh(core_axis_name="c", subcore_axis_name="s")


def _sc_wid():
    return lax.axis_index("c") * 16 + lax.axis_index("s")


@functools.partial(
    pl.kernel,
    mesh=_MESH,
    compiler_params=pltpu.CompilerParams(needs_layout_passes=False),
    out_type=[jax.ShapeDtypeStruct((_N * _K,), jnp.int32),
              jax.ShapeDtypeStruct((_NTILES * _K,), jnp.int32)],
    scratch_types=[
        pltpu.VMEM((_N * _K,), jnp.int32),
        pltpu.VMEM((_N,), jnp.int32),
        pltpu.VMEM((_RPT * _K,), jnp.int32),
        pltpu.VMEM((_K,), jnp.int32),
    ],
)
def _sc_edge_state(topk_hbm, state_hbm, part_hbm, topk_v, fl_v, st_v, pc_v):
    wid = _sc_wid()
    row0 = wid * _RPT
    pltpu.sync_copy(topk_hbm, topk_v)
    zeros = jnp.zeros((_K,), jnp.int32)
    ones = jnp.full((_K,), 1, jnp.int32)

    def zflag_body(z, _):
        fl_v[pl.ds(z * _K, _K)] = zeros
        return 0

    lax.fori_loop(0, _N // _K, zflag_body, 0)

    def row_body(r, _):
        i = row0 + r
        ivec = jnp.full((_K,), 1, jnp.int32) * i
        a = topk_v[pl.ds(i * _K, _K)]
        valid = a != ivec
        plsc.store_scatter(fl_v, [a], ones, mask=valid)
        cnt = jnp.zeros((_K,), jnp.int32)
        for u in range(_K):
            g = plsc.load_gather(topk_v, [a * _K + u])
            bit = plsc.load_gather(fl_v, [g])
            cnt = cnt + jnp.where(g != a, bit, 0)
        plsc.store_scatter(fl_v, [a], zeros, mask=valid)
        state = jnp.where(valid,
                          1 + (cnt >= 2).astype(jnp.int32),
                          jnp.zeros((_K,), jnp.int32))
        st_v[pl.ds(r * _K, _K)] = state
        pc_v[...] = pc_v[...] + (state == 2).astype(jnp.int32)
        return 0

    pc_v[...] = jnp.zeros((_K,), jnp.int32)
    lax.fori_loop(0, _RPT, row_body, 0)
    pltpu.sync_copy(st_v, state_hbm.at[pl.ds(row0 * _K, _RPT * _K)])
    pltpu.sync_copy(pc_v, part_hbm.at[pl.ds(wid * _K, _K)])


_CN = _CHUNK * _N
_NCHUNK = _RPT // _CHUNK


@functools.partial(
    pl.kernel,
    mesh=_MESH,
    compiler_params=pltpu.CompilerParams(needs_layout_passes=False),
    out_type=jax.ShapeDtypeStruct((_N * _N,), jnp.float32),
    scratch_types=[
        pltpu.VMEM((_RPT * _K,), jnp.int32),
        pltpu.VMEM((_N * _K,), jnp.int32),
        pltpu.VMEM((_NTILES * _K,), jnp.int32),
        pltpu.VMEM((32,), jnp.float32),
        pltpu.VMEM((2 * _CN,), jnp.float32),
        pltpu.SemaphoreType.DMA,
    ],
)
def _sc_scatter_adj(topk_hbm, state_hbm, part_hbm, tbl_hbm, a_hbm,
                    topk_v, stf_v, part_v, tbl_v, buf_v, sem):
    wid = _sc_wid()
    row0 = wid * _RPT
    pltpu.sync_copy(topk_hbm.at[pl.ds(row0 * _K, _RPT * _K)], topk_v)
    pltpu.sync_copy(state_hbm, stf_v)
    pltpu.sync_copy(part_hbm, part_v)
    pltpu.sync_copy(tbl_hbm, tbl_v)

    s_acc = jnp.zeros((_K,), jnp.int32)
    for t in range(_NTILES):
        s_acc = s_acc + part_v[pl.ds(t * _K, _K)]
    thr = jnp.where(jnp.sum(s_acc) >= 2 * _N, 2, 1)
    thrv = jnp.full((_K,), 1, jnp.int32) * thr
    ones = jnp.full((_K,), 1, jnp.int32)
    lane0 = lax.broadcasted_iota(jnp.int32, (_K,), 0) == 0
    zeros = jnp.zeros((_K,), jnp.float32)

    def zero_body(z, _):
        for z8 in range(16):
            buf_v[pl.ds(z * (16 * _K) + z8 * _K, _K)] = zeros
        return 0

    lax.fori_loop(0, 2 * _CN // (16 * _K), zero_body, 0)

    def scatter_row(lr, off, write):
        i = row0 + lr
        ivec = ones * i
        a = topk_v[pl.ds(lr * _K, _K)]
        sti = stf_v[pl.ds(i * _K, _K)]
        keep = jnp.logical_and(sti >= thrv, a != ivec)
        if write:
            dega = jnp.zeros((_K,), jnp.int32)
            for u in range(_K):
                s_u = plsc.load_gather(stf_v, [a * _K + u])
                dega = dega + (s_u >= thrv).astype(jnp.int32)
            degi = jnp.sum(keep.astype(jnp.int32))
            divec = plsc.load_gather(tbl_v, [ones * degi + 1])
            da = plsc.load_gather(tbl_v, [dega + 1])
            plsc.store_scatter(buf_v, [a + off], divec * da, mask=keep)
            plsc.store_scatter(buf_v, [ivec + off], divec * divec, mask=lane0)
        else:
            plsc.store_scatter(buf_v, [a + off], zeros, mask=keep)
            plsc.store_scatter(buf_v, [ivec + off], zeros, mask=lane0)

    def chunk_body(c, _):
        p = lax.rem(c, 2)
        poff = p * _CN

        @pl.when(c >= 2)
        def _wait_and_clear():
            pltpu.make_async_copy(
                buf_v.at[pl.ds(poff, _CN)],
                a_hbm.at[pl.ds((row0 + (c - 2) * _CHUNK) * _N, _CN)],
                sem).wait()
            for r in range(_CHUNK):
                scatter_row((c - 2) * _CHUNK + r, poff + r * _N, False)

        for r in range(_CHUNK):
            scatter_row(c * _CHUNK + r, poff + r * _N, True)
        pltpu.async_copy(
            buf_v.at[pl.ds(poff, _CN)],
            a_hbm.at[pl.ds((row0 + c * _CHUNK) * _N, _CN)],
            sem)
        return 0

    lax.fori_loop(0, _NCHUNK, chunk_body, 0)
    for cc in range(_NCHUNK - 2, _NCHUNK):
        pltpu.make_async_copy(
            buf_v.at[pl.ds((cc % 2) * _CN, _CN)],
            a_hbm.at[pl.ds((row0 + cc * _CHUNK) * _N, _CN)],
            sem).wait()


def kernel(x, W1, b1, W2, b2):
    n, din = x.shape
    hid = W1.shape[0]
    dout = W2.shape[0]
    nb = n // _RB

    feats = pl.pallas_call(
        _normalize_body,
        grid=(nb,),
        in_specs=[pl.BlockSpec((_RB, din), lambda i: (i, 0))],
        out_specs=pl.BlockSpec((_RB, din), lambda i: (i, 0)),
        out_shape=jax.ShapeDtypeStruct((n, din), jnp.float32),
    )(x)

    last = nb - 1
    topk, y1 = pl.pallas_call(
        _simtopk_body,
        grid=(nb + 1,),
        in_specs=[pl.BlockSpec((_RB, din), lambda i: (jnp.minimum(i, last), 0)),
                  pl.BlockSpec((n, din), lambda i: (0, 0)),
                  pl.BlockSpec((_RB, din), lambda i: (jnp.minimum(i, last), 0)),
                  pl.BlockSpec((hid, din), lambda i: (0, 0))],
        out_specs=[pl.BlockSpec((_RB, _K),
                                lambda i: (jnp.maximum(i - 1, 0), 0)),
                   pl.BlockSpec((_RB, hid),
                                lambda i: (jnp.minimum(i, last), 0))],
        out_shape=[jax.ShapeDtypeStruct((n, _K), jnp.int32),
                   jax.ShapeDtypeStruct((n, hid), jnp.float32)],
        scratch_shapes=[pltpu.VMEM((2, _RB, _N), jnp.float32)],
    )(feats, feats, x, W1)

    topk_flat = topk.reshape(n * _K)
    state_flat, part = _sc_edge_state(topk_flat)

    a_flat = _sc_scatter_adj(topk_flat, state_flat, part, _DTBL)
    a_norm = a_flat.reshape(n, n)

    y2 = pl.pallas_call(
        _layer_body,
        grid=(nb,),
        in_specs=[pl.BlockSpec((_RB, _N), lambda i: (i, 0)),
                  pl.BlockSpec((n, hid), lambda i: (0, 0)),
                  pl.BlockSpec((1, hid), lambda i: (0, 0)),
                  pl.BlockSpec((dout, hid), lambda i: (0, 0))],
        out_specs=pl.BlockSpec((_RB, dout), lambda i: (i, 0)),
        out_shape=jax.ShapeDtypeStruct((n, dout), jnp.float32),
    )(a_norm, y1, b1.reshape(1, hid), W2)

    out = pl.pallas_call(
        _out_body,
        grid=(nb,),
        in_specs=[pl.BlockSpec((_RB, _N), lambda i: (i, 0)),
                  pl.BlockSpec((n, dout), lambda i: (0, 0)),
                  pl.BlockSpec((1, dout), lambda i: (0, 0))],
        out_specs=pl.BlockSpec((_RB, dout), lambda i: (i, 0)),
        out_shape=jax.ShapeDtypeStruct((n, dout), jnp.float32),
    )(a_norm, y2, b2.reshape(1, dout))

    return out


# speedup vs baseline: 1.0597x; 1.0328x over previous
import functools

import numpy as np

import jax
import jax.numpy as jnp
from jax import lax
from jax.experimental import pallas as pl
from jax.experimental.pallas import tpu as pltpu
from jax.experimental.pallas import tpu_sc as plsc

_N = 2048
_K = 16
_RB = 256
_NEG = -3.0e38

_NTILES = 32
_RPT = _N // _NTILES
_CHUNK = 8

_DTBL = (np.maximum(np.arange(32), 1).astype(np.float64) ** -0.5
         ).astype(np.float32)


def _normalize_body(x_ref, o_ref):
    x = x_ref[...]
    norms = jnp.sqrt(jnp.sum(x * x, axis=1, keepdims=True))
    o_ref[...] = x / jnp.maximum(norms, 1e-12)


def _topk_extract(scr, out_ref):
    v = scr[...]
    col = lax.broadcasted_iota(jnp.int32, v.shape, 1)
    outs = []
    for _ in range(_K):
        m = jnp.max(v, axis=1, keepdims=True)
        eq = v == m
        outs.append(jnp.min(jnp.where(eq, col, _N), axis=1, keepdims=True))
        v = jnp.where(eq, _NEG, v)
    out_ref[...] = jnp.concatenate(outs, axis=1)


def _simtopk_body(fa_ref, fb_ref, feats_ref, ide_ref, ido_ref, scra, scrb):
    s = pl.program_id(0)
    tail = pl.num_programs(0) - 1

    @pl.when(s < tail)
    def _main():
        feats = feats_ref[...]
        _topk_extract(scrb, ido_ref)
        scra[...] = lax.dot_general(fa_ref[...], feats,
                                    (((1,), (1,)), ((), ())),
                                    preferred_element_type=jnp.float32)
        _topk_extract(scra, ide_ref)
        scrb[...] = lax.dot_general(fb_ref[...], feats,
                                    (((1,), (1,)), ((), ())),
                                    preferred_element_type=jnp.float32)

    @pl.when(s == tail)
    def _last():
        _topk_extract(scrb, ido_ref)


def _xw_body(xb_ref, w_ref, o_ref):
    o_ref[...] = lax.dot_general(xb_ref[...], w_ref[...],
                                 (((1,), (1,)), ((), ())),
                                 preferred_element_type=jnp.float32)


def _layer_body(ab_ref, y_ref, b_ref, w2_ref, o_ref):
    s = lax.dot_general(ab_ref[...], y_ref[...], (((1,), (0,)), ((), ())),
                        preferred_element_type=jnp.float32)
    h = jnp.maximum(s + b_ref[...], 0.0)
    o_ref[...] = lax.dot_general(h, w2_ref[...], (((1,), (1,)), ((), ())),
                                 preferred_element_type=jnp.float32)


def _out_body(ab_ref, y2_ref, b2_ref, o_ref):
    s = lax.dot_general(ab_ref[...], y2_ref[...], (((1,), (0,)), ((), ())),
                        preferred_element_type=jnp.float32)
    o_ref[...] = s + b2_ref[...]


_MESH = plsc.VectorSubcoreMesh(core_axis_name="c", subcore_axis_name="s")


def _sc_wid():
    return lax.axis_index("c") * 16 + lax.axis_index("s")


@functools.partial(
    pl.kernel,
    mesh=_MESH,
    compiler_params=pltpu.CompilerParams(needs_layout_passes=False),
    out_type=[jax.ShapeDtypeStruct((_N * _K,), jnp.int32),
              jax.ShapeDtypeStruct((_NTILES * _K,), jnp.int32)],
    scratch_types=[
        pltpu.VMEM((_N * _K,), jnp.int32),
        pltpu.VMEM((_N,), jnp.int32),
        pltpu.VMEM((_RPT * _K,), jnp.int32),
        pltpu.VMEM((_K,), jnp.int32),
    ],
)
def _sc_edge_state(topk_hbm, state_hbm, part_hbm, topk_v, fl_v, st_v, pc_v):
    wid = _sc_wid()
    row0 = wid * _RPT
    pltpu.sync_copy(topk_hbm, topk_v)
    zeros = jnp.zeros((_K,), jnp.int32)
    ones = jnp.full((_K,), 1, jnp.int32)

    def zflag_body(z, _):
        fl_v[pl.ds(z * _K, _K)] = zeros
        return 0

    lax.fori_loop(0, _N // _K, zflag_body, 0)

    def row_body(r, _):
        i = row0 + r
        ivec = jnp.full((_K,), 1, jnp.int32) * i
        a = topk_v[pl.ds(i * _K, _K)]
        valid = a != ivec
        plsc.store_scatter(fl_v, [a], ones, mask=valid)
        cnt = jnp.zeros((_K,), jnp.int32)
        for u in range(_K):
            g = plsc.load_gather(topk_v, [a * _K + u])
            bit = plsc.load_gather(fl_v, [g])
            cnt = cnt + jnp.where(g != a, bit, 0)
        plsc.store_scatter(fl_v, [a], zeros, mask=valid)
        state = jnp.where(valid,
                          1 + (cnt >= 2).astype(jnp.int32),
                          jnp.zeros((_K,), jnp.int32))
        st_v[pl.ds(r * _K, _K)] = state
        pc_v[...] = pc_v[...] + (state == 2).astype(jnp.int32)
        return 0

    pc_v[...] = jnp.zeros((_K,), jnp.int32)
    lax.fori_loop(0, _RPT, row_body, 0)
    pltpu.sync_copy(st_v, state_hbm.at[pl.ds(row0 * _K, _RPT * _K)])
    pltpu.sync_copy(pc_v, part_hbm.at[pl.ds(wid * _K, _K)])


_CN = _CHUNK * _N
_NCHUNK = _RPT // _CHUNK


@functools.partial(
    pl.kernel,
    mesh=_MESH,
    compiler_params=pltpu.CompilerParams(needs_layout_passes=False),
    out_type=jax.ShapeDtypeStruct((_N * _N,), jnp.float32),
    scratch_types=[
        pltpu.VMEM((_RPT * _K,), jnp.int32),
        pltpu.VMEM((_N * _K,), jnp.int32),
        pltpu.VMEM((_NTILES * _K,), jnp.int32),
        pltpu.VMEM((32,), jnp.float32),
        pltpu.VMEM((2 * _CN,), jnp.float32),
        pltpu.SemaphoreType.DMA,
    ],
)
def _sc_scatter_adj(topk_hbm, state_hbm, part_hbm, tbl_hbm, a_hbm,
                    topk_v, stf_v, part_v, tbl_v, buf_v, sem):
    wid = _sc_wid()
    row0 = wid * _RPT
    pltpu.sync_copy(topk_hbm.at[pl.ds(row0 * _K, _RPT * _K)], topk_v)
    pltpu.sync_copy(state_hbm, stf_v)
    pltpu.sync_copy(part_hbm, part_v)
    pltpu.sync_copy(tbl_hbm, tbl_v)

    s_acc = jnp.zeros((_K,), jnp.int32)
    for t in range(_NTILES):
        s_acc = s_acc + part_v[pl.ds(t * _K, _K)]
    thr = jnp.where(jnp.sum(s_acc) >= 2 * _N, 2, 1)
    thrv = jnp.full((_K,), 1, jnp.int32) * thr
    ones = jnp.full((_K,), 1, jnp.int32)
    lane0 = lax.broadcasted_iota(jnp.int32, (_K,), 0) == 0
    zeros = jnp.zeros((_K,), jnp.float32)

    def zero_body(z, _):
        for z8 in range(16):
            buf_v[pl.ds(z * (16 * _K) + z8 * _K, _K)] = zeros
        return 0

    lax.fori_loop(0, 2 * _CN // (16 * _K), zero_body, 0)

    def scatter_row(lr, off, write):
        i = row0 + lr
        ivec = ones * i
        a = topk_v[pl.ds(lr * _K, _K)]
        sti = stf_v[pl.ds(i * _K, _K)]
        keep = jnp.logical_and(sti >= thrv, a != ivec)
        if write:
            dega = jnp.zeros((_K,), jnp.int32)
            for u in range(_K):
                s_u = plsc.load_gather(stf_v, [a * _K + u])
                dega = dega + (s_u >= thrv).astype(jnp.int32)
            degi = jnp.sum(keep.astype(jnp.int32))
            divec = plsc.load_gather(tbl_v, [ones * degi + 1])
            da = plsc.load_gather(tbl_v, [dega + 1])
            plsc.store_scatter(buf_v, [a + off], divec * da, mask=keep)
            plsc.store_scatter(buf_v, [ivec + off], divec * divec, mask=lane0)
        else:
            plsc.store_scatter(buf_v, [a + off], zeros, mask=keep)
            plsc.store_scatter(buf_v, [ivec + off], zeros, mask=lane0)

    def chunk_body(c, _):
        p = lax.rem(c, 2)
        poff = p * _CN

        @pl.when(c >= 2)
        def _wait_and_clear():
            pltpu.make_async_copy(
                buf_v.at[pl.ds(poff, _CN)],
                a_hbm.at[pl.ds((row0 + (c - 2) * _CHUNK) * _N, _CN)],
                sem).wait()
            for r in range(_CHUNK):
                scatter_row((c - 2) * _CHUNK + r, poff + r * _N, False)

        for r in range(_CHUNK):
            scatter_row(c * _CHUNK + r, poff + r * _N, True)
        pltpu.async_copy(
            buf_v.at[pl.ds(poff, _CN)],
            a_hbm.at[pl.ds((row0 + c * _CHUNK) * _N, _CN)],
            sem)
        return 0

    lax.fori_loop(0, _NCHUNK, chunk_body, 0)
    for cc in range(_NCHUNK - 2, _NCHUNK):
        pltpu.make_async_copy(
            buf_v.at[pl.ds((cc % 2) * _CN, _CN)],
            a_hbm.at[pl.ds((row0 + cc * _CHUNK) * _N, _CN)],
            sem).wait()


def kernel(x, W1, b1, W2, b2):
    n, din = x.shape
    hid = W1.shape[0]
    dout = W2.shape[0]
    nb = n // _RB

    feats = pl.pallas_call(
        _normalize_body,
        grid=(nb,),
        in_specs=[pl.BlockSpec((_RB, din), lambda i: (i, 0))],
        out_specs=pl.BlockSpec((_RB, din), lambda i: (i, 0)),
        out_shape=jax.ShapeDtypeStruct((n, din), jnp.float32),
    )(x)

    nh = nb // 2
    ide, ido = pl.pallas_call(
        _simtopk_body,
        grid=(nh + 1,),
        in_specs=[
            pl.BlockSpec((_RB, din),
                         lambda s: (jnp.minimum(2 * s, 2 * nh - 2), 0)),
            pl.BlockSpec((_RB, din),
                         lambda s: (jnp.minimum(2 * s + 1, 2 * nh - 1), 0)),
            pl.BlockSpec((n, din), lambda s: (0, 0)),
        ],
        out_specs=[
            pl.BlockSpec((_RB, _K), lambda s: (jnp.minimum(s, nh - 1), 0)),
            pl.BlockSpec((_RB, _K), lambda s: (jnp.maximum(s - 1, 0), 0)),
        ],
        out_shape=[jax.ShapeDtypeStruct((n // 2, _K), jnp.int32),
                   jax.ShapeDtypeStruct((n // 2, _K), jnp.int32)],
        scratch_shapes=[pltpu.VMEM((_RB, _N), jnp.float32),
                        pltpu.VMEM((_RB, _N), jnp.float32)],
    )(feats, feats, feats)
    topk = jnp.stack([ide.reshape(nh, _RB, _K), ido.reshape(nh, _RB, _K)],
                     axis=1).reshape(n, _K)

    y1 = pl.pallas_call(
        _xw_body,
        grid=(nb,),
        in_specs=[pl.BlockSpec((_RB, din), lambda i: (i, 0)),
                  pl.BlockSpec((hid, din), lambda i: (0, 0))],
        out_specs=pl.BlockSpec((_RB, hid), lambda i: (i, 0)),
        out_shape=jax.ShapeDtypeStruct((n, hid), jnp.float32),
    )(x, W1)

    topk_flat = topk.reshape(n * _K)
    state_flat, part = _sc_edge_state(topk_flat)

    a_flat = _sc_scatter_adj(topk_flat, state_flat, part, _DTBL)
    a_norm = a_flat.reshape(n, n)

    y2 = pl.pallas_call(
        _layer_body,
        grid=(nb,),
        in_specs=[pl.BlockSpec((_RB, _N), lambda i: (i, 0)),
                  pl.BlockSpec((n, hid), lambda i: (0, 0)),
                  pl.BlockSpec((1, hid), lambda i: (0, 0)),
                  pl.BlockSpec((dout, hid), lambda i: (0, 0))],
        out_specs=pl.BlockSpec((_RB, dout), lambda i: (i, 0)),
        out_shape=jax.ShapeDtypeStruct((n, dout), jnp.float32),
    )(a_norm, y1, b1.reshape(1, hid), W2)

    out = pl.pallas_call(
        _out_body,
        grid=(nb,),
        in_specs=[pl.BlockSpec((_RB, _N), lambda i: (i, 0)),
                  pl.BlockSpec((n, dout), lambda i: (0, 0)),
                  pl.BlockSpec((1, dout), lambda i: (0, 0))],
        out_specs=pl.BlockSpec((_RB, dout), lambda i: (i, 0)),
        out_shape=jax.ShapeDtypeStruct((n, dout), jnp.float32),
    )(a_norm, y2, b2.reshape(1, dout))

    return out


# speedup vs baseline: 1.0638x; 1.0039x over previous
import functools

import numpy as np

import jax
import jax.numpy as jnp
from jax import lax
from jax.experimental import pallas as pl
from jax.experimental.pallas import tpu as pltpu
from jax.experimental.pallas import tpu_sc as plsc

_N = 2048
_K = 16
_RB = 256
_NEG = -3.0e38

_NTILES = 32
_RPT = _N // _NTILES
_CHUNK = 8

_DTBL = (np.maximum(np.arange(32), 1).astype(np.float64) ** -0.5
         ).astype(np.float32)


def _normalize_body(x_ref, o_ref):
    x = x_ref[...]
    norms = jnp.sqrt(jnp.sum(x * x, axis=1, keepdims=True))
    o_ref[...] = x / jnp.maximum(norms, 1e-12)


def _topk_extract(scr, out_ref):
    v = scr[...]
    col = lax.broadcasted_iota(jnp.int32, v.shape, 1)
    outs = []
    for _ in range(_K):
        m = jnp.max(v, axis=1, keepdims=True)
        eq = v == m
        outs.append(jnp.min(jnp.where(eq, col, _N), axis=1, keepdims=True))
        v = jnp.where(eq, _NEG, v)
    out_ref[...] = jnp.concatenate(outs, axis=1)


def _simtopk_body(fa_ref, fb_ref, feats_ref, ide_ref, ido_ref, scra, scrb):
    s = pl.program_id(0)
    tail = pl.num_programs(0) - 1

    @pl.when(s < tail)
    def _main():
        feats = feats_ref[...]
        _topk_extract(scrb, ido_ref)
        scra[...] = lax.dot_general(fa_ref[...], feats,
                                    (((1,), (1,)), ((), ())),
                                    preferred_element_type=jnp.float32)
        _topk_extract(scra, ide_ref)
        scrb[...] = lax.dot_general(fb_ref[...], feats,
                                    (((1,), (1,)), ((), ())),
                                    preferred_element_type=jnp.float32)

    @pl.when(s == tail)
    def _last():
        _topk_extract(scrb, ido_ref)


def _xw_body(xb_ref, w_ref, o_ref):
    o_ref[...] = lax.dot_general(xb_ref[...], w_ref[...],
                                 (((1,), (1,)), ((), ())),
                                 preferred_element_type=jnp.float32
                                 ).astype(jnp.bfloat16)


def _layer_body(ab_ref, y_ref, b_ref, w2_ref, o_ref):
    s = lax.dot_general(ab_ref[...], y_ref[...], (((1,), (0,)), ((), ())),
                        preferred_element_type=jnp.float32)
    h = jnp.maximum(s + b_ref[...], 0.0)
    o_ref[...] = lax.dot_general(h, w2_ref[...], (((1,), (1,)), ((), ())),
                                 preferred_element_type=jnp.float32
                                 ).astype(jnp.bfloat16)


def _out_body(ab_ref, y2_ref, b2_ref, o_ref):
    s = lax.dot_general(ab_ref[...], y2_ref[...], (((1,), (0,)), ((), ())),
                        preferred_element_type=jnp.float32)
    o_ref[...] = s + b2_ref[...]


_MESH = plsc.VectorSubcoreMesh(core_axis_name="c", subcore_axis_name="s")


def _sc_wid():
    return lax.axis_index("c") * 16 + lax.axis_index("s")


@functools.partial(
    pl.kernel,
    mesh=_MESH,
    compiler_params=pltpu.CompilerParams(needs_layout_passes=False),
    out_type=[jax.ShapeDtypeStruct((_N * _K,), jnp.int32),
              jax.ShapeDtypeStruct((_NTILES * _K,), jnp.int32)],
    scratch_types=[
        pltpu.VMEM((_N * _K,), jnp.int32),
        pltpu.VMEM((_N,), jnp.int32),
        pltpu.VMEM((_RPT * _K,), jnp.int32),
        pltpu.VMEM((_K,), jnp.int32),
    ],
)
def _sc_edge_state(topk_hbm, state_hbm, part_hbm, topk_v, fl_v, st_v, pc_v):
    wid = _sc_wid()
    row0 = wid * _RPT
    pltpu.sync_copy(topk_hbm, topk_v)
    zeros = jnp.zeros((_K,), jnp.int32)
    ones = jnp.full((_K,), 1, jnp.int32)

    def zflag_body(z, _):
        fl_v[pl.ds(z * _K, _K)] = zeros
        return 0

    lax.fori_loop(0, _N // _K, zflag_body, 0)

    def row_body(r, _):
        i = row0 + r
        ivec = jnp.full((_K,), 1, jnp.int32) * i
        a = topk_v[pl.ds(i * _K, _K)]
        valid = a != ivec
        plsc.store_scatter(fl_v, [a], ones, mask=valid)
        cnt = jnp.zeros((_K,), jnp.int32)
        for u in range(_K):
            g = plsc.load_gather(topk_v, [a * _K + u])
            bit = plsc.load_gather(fl_v, [g])
            cnt = cnt + jnp.where(g != a, bit, 0)
        plsc.store_scatter(fl_v, [a], zeros, mask=valid)
        state = jnp.where(valid,
                          1 + (cnt >= 2).astype(jnp.int32),
                          jnp.zeros((_K,), jnp.int32))
        st_v[pl.ds(r * _K, _K)] = state
        pc_v[...] = pc_v[...] + (state == 2).astype(jnp.int32)
        return 0

    pc_v[...] = jnp.zeros((_K,), jnp.int32)
    lax.fori_loop(0, _RPT, row_body, 0)
    pltpu.sync_copy(st_v, state_hbm.at[pl.ds(row0 * _K, _RPT * _K)])
    pltpu.sync_copy(pc_v, part_hbm.at[pl.ds(wid * _K, _K)])


_CN = _CHUNK * _N
_NCHUNK = _RPT // _CHUNK


@functools.partial(
    pl.kernel,
    mesh=_MESH,
    compiler_params=pltpu.CompilerParams(needs_layout_passes=False),
    out_type=jax.ShapeDtypeStruct((_N * _N,), jnp.float32),
    scratch_types=[
        pltpu.VMEM((_RPT * _K,), jnp.int32),
        pltpu.VMEM((_N * _K,), jnp.int32),
        pltpu.VMEM((_NTILES * _K,), jnp.int32),
        pltpu.VMEM((32,), jnp.float32),
        pltpu.VMEM((2 * _CN,), jnp.float32),
        pltpu.SemaphoreType.DMA,
    ],
)
def _sc_scatter_adj(topk_hbm, state_hbm, part_hbm, tbl_hbm, a_hbm,
                    topk_v, stf_v, part_v, tbl_v, buf_v, sem):
    wid = _sc_wid()
    row0 = wid * _RPT
    pltpu.sync_copy(topk_hbm.at[pl.ds(row0 * _K, _RPT * _K)], topk_v)
    pltpu.sync_copy(state_hbm, stf_v)
    pltpu.sync_copy(part_hbm, part_v)
    pltpu.sync_copy(tbl_hbm, tbl_v)

    s_acc = jnp.zeros((_K,), jnp.int32)
    for t in range(_NTILES):
        s_acc = s_acc + part_v[pl.ds(t * _K, _K)]
    thr = jnp.where(jnp.sum(s_acc) >= 2 * _N, 2, 1)
    thrv = jnp.full((_K,), 1, jnp.int32) * thr
    ones = jnp.full((_K,), 1, jnp.int32)
    lane0 = lax.broadcasted_iota(jnp.int32, (_K,), 0) == 0
    zeros = jnp.zeros((_K,), jnp.float32)

    def zero_body(z, _):
        for z8 in range(16):
            buf_v[pl.ds(z * (16 * _K) + z8 * _K, _K)] = zeros
        return 0

    lax.fori_loop(0, 2 * _CN // (16 * _K), zero_body, 0)

    def scatter_row(lr, off, write):
        i = row0 + lr
        ivec = ones * i
        a = topk_v[pl.ds(lr * _K, _K)]
        sti = stf_v[pl.ds(i * _K, _K)]
        keep = jnp.logical_and(sti >= thrv, a != ivec)
        if write:
            dega = jnp.zeros((_K,), jnp.int32)
            for u in range(_K):
                s_u = plsc.load_gather(stf_v, [a * _K + u])
                dega = dega + (s_u >= thrv).astype(jnp.int32)
            degi = jnp.sum(keep.astype(jnp.int32))
            divec = plsc.load_gather(tbl_v, [ones * degi + 1])
            da = plsc.load_gather(tbl_v, [dega + 1])
            plsc.store_scatter(buf_v, [a + off], divec * da, mask=keep)
            plsc.store_scatter(buf_v, [ivec + off], divec * divec, mask=lane0)
        else:
            plsc.store_scatter(buf_v, [a + off], zeros, mask=keep)
            plsc.store_scatter(buf_v, [ivec + off], zeros, mask=lane0)

    def chunk_body(c, _):
        p = lax.rem(c, 2)
        poff = p * _CN

        @pl.when(c >= 2)
        def _wait_and_clear():
            pltpu.make_async_copy(
                buf_v.at[pl.ds(poff, _CN)],
                a_hbm.at[pl.ds((row0 + (c - 2) * _CHUNK) * _N, _CN)],
                sem).wait()
            for r in range(_CHUNK):
                scatter_row((c - 2) * _CHUNK + r, poff + r * _N, False)

        for r in range(_CHUNK):
            scatter_row(c * _CHUNK + r, poff + r * _N, True)
        pltpu.async_copy(
            buf_v.at[pl.ds(poff, _CN)],
            a_hbm.at[pl.ds((row0 + c * _CHUNK) * _N, _CN)],
            sem)
        return 0

    lax.fori_loop(0, _NCHUNK, chunk_body, 0)
    for cc in range(_NCHUNK - 2, _NCHUNK):
        pltpu.make_async_copy(
            buf_v.at[pl.ds((cc % 2) * _CN, _CN)],
            a_hbm.at[pl.ds((row0 + cc * _CHUNK) * _N, _CN)],
            sem).wait()


def kernel(x, W1, b1, W2, b2):
    n, din = x.shape
    hid = W1.shape[0]
    dout = W2.shape[0]
    nb = n // _RB

    feats = pl.pallas_call(
        _normalize_body,
        grid=(nb,),
        in_specs=[pl.BlockSpec((_RB, din), lambda i: (i, 0))],
        out_specs=pl.BlockSpec((_RB, din), lambda i: (i, 0)),
        out_shape=jax.ShapeDtypeStruct((n, din), jnp.float32),
    )(x)

    nh = nb // 2
    ide, ido = pl.pallas_call(
        _simtopk_body,
        grid=(nh + 1,),
        in_specs=[
            pl.BlockSpec((_RB, din),
                         lambda s: (jnp.minimum(2 * s, 2 * nh - 2), 0)),
            pl.BlockSpec((_RB, din),
                         lambda s: (jnp.minimum(2 * s + 1, 2 * nh - 1), 0)),
            pl.BlockSpec((n, din), lambda s: (0, 0)),
        ],
        out_specs=[
            pl.BlockSpec((_RB, _K), lambda s: (jnp.minimum(s, nh - 1), 0)),
            pl.BlockSpec((_RB, _K), lambda s: (jnp.maximum(s - 1, 0), 0)),
        ],
        out_shape=[jax.ShapeDtypeStruct((n // 2, _K), jnp.int32),
                   jax.ShapeDtypeStruct((n // 2, _K), jnp.int32)],
        scratch_shapes=[pltpu.VMEM((_RB, _N), jnp.float32),
                        pltpu.VMEM((_RB, _N), jnp.float32)],
    )(feats, feats, feats)
    topk = jnp.stack([ide.reshape(nh, _RB, _K), ido.reshape(nh, _RB, _K)],
                     axis=1).reshape(n, _K)

    y1 = pl.pallas_call(
        _xw_body,
        grid=(nb,),
        in_specs=[pl.BlockSpec((_RB, din), lambda i: (i, 0)),
                  pl.BlockSpec((hid, din), lambda i: (0, 0))],
        out_specs=pl.BlockSpec((_RB, hid), lambda i: (i, 0)),
        out_shape=jax.ShapeDtypeStruct((n, hid), jnp.bfloat16),
    )(x, W1)

    topk_flat = topk.reshape(n * _K)
    state_flat, part = _sc_edge_state(topk_flat)

    a_flat = _sc_scatter_adj(topk_flat, state_flat, part, _DTBL)
    a_norm = a_flat.reshape(n, n).astype(jnp.bfloat16)

    y2 = pl.pallas_call(
        _layer_body,
        grid=(nb,),
        in_specs=[pl.BlockSpec((_RB, _N), lambda i: (i, 0)),
                  pl.BlockSpec((n, hid), lambda i: (0, 0)),
                  pl.BlockSpec((1, hid), lambda i: (0, 0)),
                  pl.BlockSpec((dout, hid), lambda i: (0, 0))],
        out_specs=pl.BlockSpec((_RB, dout), lambda i: (i, 0)),
        out_shape=jax.ShapeDtypeStruct((n, dout), jnp.bfloat16),
    )(a_norm, y1, b1.reshape(1, hid), W2)

    out = pl.pallas_call(
        _out_body,
        grid=(nb,),
        in_specs=[pl.BlockSpec((_RB, _N), lambda i: (i, 0)),
                  pl.BlockSpec((n, dout), lambda i: (0, 0)),
                  pl.BlockSpec((1, dout), lambda i: (0, 0))],
        out_specs=pl.BlockSpec((_RB, dout), lambda i: (i, 0)),
        out_shape=jax.ShapeDtypeStruct((n, dout), jnp.float32),
    )(a_norm, y2, b2.reshape(1, dout))

    return out
